# Initial kernel scaffold; baseline (speedup 1.0000x reference)
import jax
import jax.numpy as jnp
from jax.experimental import pallas as pl


def kernel(node_pos, areas, edges, info, params):
    raise NotImplementedError("write your pallas kernel here")


# speedup vs baseline: 1442.1681x; 1442.1681x over previous
import functools

import jax
import jax.numpy as jnp
from jax.experimental import pallas as pl

N_NODES = 10000
N_EDGES = 160000
F = 128
LN_EPS = 1e-5

_NODE_TILE = 2000
_EDGE_TILE = 2000


def _dot(a, b):
    return jax.lax.dot_general(
        a, b, (((1,), (0,)), ((), ())), preferred_element_type=jnp.float32
    )


def _full(shape):
    return pl.BlockSpec(shape, lambda i: (0,) * len(shape))


def _rows(tile, width):
    return pl.BlockSpec((tile, width), lambda i: (i, 0))


def _enc_body(x_ref, w1_ref, b1_ref, w2_ref, b2_ref, w3_ref, b3_ref,
              g_ref, bb_ref, o_ref):
    h = jnp.maximum(_dot(x_ref[...], w1_ref[...]) + b1_ref[...], 0.0)
    h = jnp.maximum(_dot(h, w2_ref[...]) + b2_ref[...], 0.0)
    h = _dot(h, w3_ref[...]) + b3_ref[...]
    mu = jnp.mean(h, axis=-1, keepdims=True)
    var = jnp.mean((h - mu) ** 2, axis=-1, keepdims=True)
    h = (h - mu) * jax.lax.rsqrt(var + LN_EPS)
    o_ref[...] = h * g_ref[...] + bb_ref[...]


def _enc_mlp(x, w1, b1, w2, b2, w3, b3, gamma, beta, tile):
    rows, k = x.shape
    grid = (rows // tile,)
    return pl.pallas_call(
        _enc_body,
        grid=grid,
        in_specs=[
            _rows(tile, k),
            _full((k, F)), _full((1, F)),
            _full((F, F)), _full((1, F)),
            _full((F, F)), _full((1, F)),
            _full((1, F)), _full((1, F)),
        ],
        out_specs=_rows(tile, F),
        out_shape=jax.ShapeDtypeStruct((rows, F), jnp.float32),
    )(x, w1, b1[None], w2, b2[None], w3, b3[None], gamma[None], beta[None])


def _edge_body(g_ref, e_ref, c_ref, b1_ref, w2_ref, b2_ref, w3_ref, b3_ref,
               ee_ref, en_ref):
    e = e_ref[...]
    h = jnp.maximum(g_ref[...] + _dot(e, c_ref[...]) + b1_ref[...], 0.0)
    h = jnp.maximum(_dot(h, w2_ref[...]) + b2_ref[...], 0.0)
    ee = _dot(h, w3_ref[...]) + b3_ref[...]
    ee_ref[...] = ee
    en_ref[...] = e + ee


def _edge_layer(G, E, c, b1, w2, b2, w3, b3):
    grid = (N_EDGES // _EDGE_TILE,)
    return pl.pallas_call(
        _edge_body,
        grid=grid,
        in_specs=[
            _rows(_EDGE_TILE, F), _rows(_EDGE_TILE, F),
            _full((F, F)), _full((1, F)),
            _full((F, F)), _full((1, F)),
            _full((F, F)), _full((1, F)),
        ],
        out_specs=[_rows(_EDGE_TILE, F), _rows(_EDGE_TILE, F)],
        out_shape=[
            jax.ShapeDtypeStruct((N_EDGES, F), jnp.float32),
            jax.ShapeDtypeStruct((N_EDGES, F), jnp.float32),
        ],
    )(G, E, c, b1[None], w2, b2[None], w3, b3[None])


def _node_body(v_ref, s_ref, na_ref, nb_ref, b1_ref, w2_ref, b2_ref,
               w3_ref, b3_ref, a_ref, bpr_ref, v_out, vs_out, vr_out):
    v = v_ref[...]
    h = jnp.maximum(
        _dot(v, na_ref[...]) + _dot(s_ref[...], nb_ref[...]) + b1_ref[...], 0.0)
    h = jnp.maximum(_dot(h, w2_ref[...]) + b2_ref[...], 0.0)
    vn = v + _dot(h, w3_ref[...]) + b3_ref[...]
    v_out[...] = vn
    vs_out[...] = _dot(vn, a_ref[...])
    vr_out[...] = _dot(vn, bpr_ref[...])


def _node_layer(V, S, na, nb, b1, w2, b2, w3, b3, a_next, b_next):
    grid = (N_NODES // _NODE_TILE,)
    return pl.pallas_call(
        _node_body,
        grid=grid,
        in_specs=[
            _rows(_NODE_TILE, F), _rows(_NODE_TILE, F),
            _full((F, F)), _full((F, F)), _full((1, F)),
            _full((F, F)), _full((1, F)),
            _full((F, F)), _full((1, F)),
            _full((F, F)), _full((F, F)),
        ],
        out_specs=[_rows(_NODE_TILE, F)] * 3,
        out_shape=[jax.ShapeDtypeStruct((N_NODES, F), jnp.float32)] * 3,
    )(V, S, na, nb, b1[None], w2, b2[None], w3, b3[None], a_next, b_next)


def _proj_body(v_ref, a_ref, b_ref, vs_out, vr_out):
    v = v_ref[...]
    vs_out[...] = _dot(v, a_ref[...])
    vr_out[...] = _dot(v, b_ref[...])


def _proj(V, a, b):
    grid = (N_NODES // _NODE_TILE,)
    return pl.pallas_call(
        _proj_body,
        grid=grid,
        in_specs=[_rows(_NODE_TILE, F), _full((F, F)), _full((F, F))],
        out_specs=[_rows(_NODE_TILE, F)] * 2,
        out_shape=[jax.ShapeDtypeStruct((N_NODES, F), jnp.float32)] * 2,
    )(V, a, b)


def _dec_body(v_ref, w1_ref, b1_ref, w2_ref, b2_ref, w3_ref, b3_ref, o_ref):
    h = jnp.maximum(_dot(v_ref[...], w1_ref[...]) + b1_ref[...], 0.0)
    h = jnp.maximum(_dot(h, w2_ref[...]) + b2_ref[...], 0.0)
    o_ref[...] = _dot(h, w3_ref[...]) + b3_ref[...]


def _dec(V, w1, b1, w2, b2, w3, b3):
    grid = (N_NODES // _NODE_TILE,)
    return pl.pallas_call(
        _dec_body,
        grid=grid,
        in_specs=[
            _rows(_NODE_TILE, F),
            _full((F, F)), _full((1, F)),
            _full((F, F)), _full((1, F)),
            _full((F, 1)), _full((1, 1)),
        ],
        out_specs=_rows(_NODE_TILE, 1),
        out_shape=jax.ShapeDtypeStruct((N_NODES, 1), jnp.float32),
    )(V, w1, b1[None], w2, b2[None], w3, b3[None])


def kernel(node_pos, areas, edges, info, params):
    pos = node_pos[0]
    ar = areas[0]
    send = edges[0, :, 0]
    recv = edges[0, :, 1]
    inf = info.reshape(-1)

    fv, fe, gnn, dec = params["fv"], params["fe"], params["gnn"], params["dec"]

    (w1v, b1v), (w2v, b2v), (w3v, b3v) = fv["layers"]
    b1v_eff = b1v + inf @ w1v[4:12]
    x4 = jnp.concatenate([pos, ar], axis=1)
    V = _enc_mlp(x4, w1v[:4], b1v_eff, w2v, b2v, w3v, b3v,
                 fv["ln"][0], fv["ln"][1], _NODE_TILE)

    ps = pos.at[send].get(mode="promise_in_bounds")
    pr = pos.at[recv].get(mode="promise_in_bounds")
    d = ps - pr
    nrm = jnp.sqrt(jnp.sum(d * d, axis=-1, keepdims=True))
    ef = jnp.concatenate([d, nrm], axis=1)
    (w1e, b1e), (w2e, b2e), (w3e, b3e) = fe["layers"]
    E = _enc_mlp(ef, w1e, b1e, w2e, b2e, w3e, b3e,
                 fe["ln"][0], fe["ln"][1], _EDGE_TILE)

    g0 = gnn[0]
    w1_0 = g0["f_edge"]["layers"][0][0]
    Vs, Vr = _proj(V, w1_0[:F], w1_0[F:2 * F])
    for l in range(15):
        g = gnn[l]
        (we1, be1), (we2, be2), (we3, be3) = g["f_edge"]["layers"]
        C = we1[2 * F:]
        G = (Vs.at[send].get(mode="promise_in_bounds")
             + Vr.at[recv].get(mode="promise_in_bounds"))
        ee, E = _edge_layer(G, E, C, be1, we2, be2, we3, be3)
        esum = jax.ops.segment_sum(ee, send, num_segments=N_NODES)
        (wn1, bn1), (wn2, bn2), (wn3, bn3) = g["f_node"]["layers"]
        if l + 1 < 15:
            w1_next = gnn[l + 1]["f_edge"]["layers"][0][0]
            a_next, b_next = w1_next[:F], w1_next[F:2 * F]
        else:
            a_next = jnp.zeros((F, F), jnp.float32)
            b_next = jnp.zeros((F, F), jnp.float32)
        V, Vs, Vr = _node_layer(V, esum, wn1[:F], wn1[F:], bn1,
                                wn2, bn2, wn3, bn3, a_next, b_next)

    (wd1, bd1), (wd2, bd2), (wd3, bd3) = dec["layers"]
    out = _dec(V, wd1, bd1, wd2, bd2, wd3, bd3)
    return out[None]


# speedup vs baseline: 2179.7250x; 1.5114x over previous
import jax
import jax.numpy as jnp
from jax import lax
from jax.experimental import pallas as pl
from jax.experimental.pallas import tpu as pltpu

N_NODES = 10000
N_EDGES = 160000
F = 128
LN_EPS = 1e-5

NP = 10240
EP = 172032
E_TILE = 2048
N_TILE = 2048
CH = 128
NCH = E_TILE // CH
E_GRID = EP // E_TILE
N_GRID = NP // N_TILE


def _dot(a, b):
    return lax.dot_general(
        a, b, (((1,), (0,)), ((), ())), preferred_element_type=jnp.float32
    )


def _dot_c0(a, b):
    return lax.dot_general(
        a, b, (((0,), (0,)), ((), ())), preferred_element_type=jnp.float32
    )


def _full(shape):
    return pl.BlockSpec(shape, lambda i: (0,) * len(shape))


def _fullp(shape):
    return pl.BlockSpec(shape, lambda i, *_: (0,) * len(shape))


def _rows(tile, width):
    return pl.BlockSpec((tile, width), lambda i: (i, 0))


def _rowsp(tile, width):
    return pl.BlockSpec((tile, width), lambda i, *_: (i, 0))


def _enc_body(x_ref, w1_ref, b1_ref, w2_ref, b2_ref, w3_ref, b3_ref,
              g_ref, bb_ref, o_ref):
    h = jnp.maximum(_dot(x_ref[...], w1_ref[...]) + b1_ref[...], 0.0)
    h = jnp.maximum(_dot(h, w2_ref[...]) + b2_ref[...], 0.0)
    h = _dot(h, w3_ref[...]) + b3_ref[...]
    mu = jnp.mean(h, axis=-1, keepdims=True)
    var = jnp.mean((h - mu) ** 2, axis=-1, keepdims=True)
    h = (h - mu) * lax.rsqrt(var + LN_EPS)
    o_ref[...] = h * g_ref[...] + bb_ref[...]


def _enc_mlp(x, w1, b1, w2, b2, w3, b3, gamma, beta, tile):
    rows, k = x.shape
    return pl.pallas_call(
        _enc_body,
        grid=(rows // tile,),
        in_specs=[
            _rows(tile, k),
            _full((k, F)), _full((1, F)),
            _full((F, F)), _full((1, F)),
            _full((F, F)), _full((1, F)),
            _full((1, F)), _full((1, F)),
        ],
        out_specs=_rows(tile, F),
        out_shape=jax.ShapeDtypeStruct((rows, F), jnp.float32),
    )(x, w1, b1[None], w2, b2[None], w3, b3[None], gamma[None], beta[None])


def _edge_body(los_ref, ss_ref, mask_ref, gr_ref, e_ref, vs_ref,
               c_ref, b1_ref, w2_ref, b2_ref, w3_ref, b3_ref,
               en_ref, esum_ref):
    t = pl.program_id(0)

    @pl.when(t == 0)
    def _init():
        esum_ref[...] = jnp.zeros_like(esum_ref)

    iota_w = lax.broadcasted_iota(jnp.int32, (CH, CH), 0)
    ohms = []
    parts = []
    for c in range(NCH):
        lo = los_ref[t * NCH + c]
        ids = ss_ref[0, :, c * CH:(c + 1) * CH]
        rel = jnp.broadcast_to(ids - lo, (CH, CH))
        oh = (rel == iota_w).astype(jnp.float32)
        mval = mask_ref[0, :, c * CH:(c + 1) * CH]
        ohms.append(oh * jnp.broadcast_to(mval, (CH, CH)))
        win = vs_ref[pl.ds(lo, CH), :]
        parts.append(_dot_c0(oh, win))
    gs = jnp.concatenate(parts, axis=0)

    e = e_ref[...]
    h = jnp.maximum(gs + gr_ref[...] + _dot(e, c_ref[...]) + b1_ref[...], 0.0)
    h = jnp.maximum(_dot(h, w2_ref[...]) + b2_ref[...], 0.0)
    ee = _dot(h, w3_ref[...]) + b3_ref[...]
    en_ref[...] = e + ee

    for c in range(NCH):
        lo = los_ref[t * NCH + c]
        contrib = _dot(ohms[c], ee[c * CH:(c + 1) * CH, :])
        esum_ref[pl.ds(lo, CH), :] += contrib


def _edge_layer(los, ss3, mask3, Gr, E, Vs, c, b1, w2, b2, w3, b3):
    grid_spec = pltpu.PrefetchScalarGridSpec(
        num_scalar_prefetch=1,
        grid=(E_GRID,),
        in_specs=[
            pl.BlockSpec((1, 1, E_TILE), lambda t, *_: (t, 0, 0)),
            pl.BlockSpec((1, 1, E_TILE), lambda t, *_: (t, 0, 0)),
            _rowsp(E_TILE, F), _rowsp(E_TILE, F),
            _fullp((NP, F)),
            _fullp((F, F)), _fullp((1, F)),
            _fullp((F, F)), _fullp((1, F)),
            _fullp((F, F)), _fullp((1, F)),
        ],
        out_specs=[_rowsp(E_TILE, F), _fullp((NP, F))],
    )
    return pl.pallas_call(
        _edge_body,
        grid_spec=grid_spec,
        out_shape=[
            jax.ShapeDtypeStruct((EP, F), jnp.float32),
            jax.ShapeDtypeStruct((NP, F), jnp.float32),
        ],
    )(los, ss3, mask3, Gr, E, Vs, c, b1[None], w2, b2[None], w3, b3[None])


def _node_body(v_ref, s_ref, na_ref, nb_ref, b1_ref, w2_ref, b2_ref,
               w3_ref, b3_ref, a_ref, bpr_ref, v_out, vs_out, vr_out):
    v = v_ref[...]
    h = jnp.maximum(
        _dot(v, na_ref[...]) + _dot(s_ref[...], nb_ref[...]) + b1_ref[...], 0.0)
    h = jnp.maximum(_dot(h, w2_ref[...]) + b2_ref[...], 0.0)
    vn = v + _dot(h, w3_ref[...]) + b3_ref[...]
    v_out[...] = vn
    vs_out[...] = _dot(vn, a_ref[...])
    vr_out[...] = _dot(vn, bpr_ref[...])


def _node_layer(V, S, na, nb, b1, w2, b2, w3, b3, a_next, b_next):
    return pl.pallas_call(
        _node_body,
        grid=(N_GRID,),
        in_specs=[
            _rows(N_TILE, F), _rows(N_TILE, F),
            _full((F, F)), _full((F, F)), _full((1, F)),
            _full((F, F)), _full((1, F)),
            _full((F, F)), _full((1, F)),
            _full((F, F)), _full((F, F)),
        ],
        out_specs=[_rows(N_TILE, F)] * 3,
        out_shape=[jax.ShapeDtypeStruct((NP, F), jnp.float32)] * 3,
    )(V, S, na, nb, b1[None], w2, b2[None], w3, b3[None], a_next, b_next)


def _proj_body(v_ref, a_ref, b_ref, vs_out, vr_out):
    v = v_ref[...]
    vs_out[...] = _dot(v, a_ref[...])
    vr_out[...] = _dot(v, b_ref[...])


def _proj(V, a, b):
    return pl.pallas_call(
        _proj_body,
        grid=(N_GRID,),
        in_specs=[_rows(N_TILE, F), _full((F, F)), _full((F, F))],
        out_specs=[_rows(N_TILE, F)] * 2,
        out_shape=[jax.ShapeDtypeStruct((NP, F), jnp.float32)] * 2,
    )(V, a, b)


def _dec_body(v_ref, w1_ref, b1_ref, w2_ref, b2_ref, w3_ref, b3_ref, o_ref):
    h = jnp.maximum(_dot(v_ref[...], w1_ref[...]) + b1_ref[...], 0.0)
    h = jnp.maximum(_dot(h, w2_ref[...]) + b2_ref[...], 0.0)
    o_ref[...] = _dot(h, w3_ref[...]) + b3_ref[...]


def _dec(V, w1, b1, w2, b2, w3, b3):
    return pl.pallas_call(
        _dec_body,
        grid=(N_GRID,),
        in_specs=[
            _rows(N_TILE, F),
            _full((F, F)), _full((1, F)),
            _full((F, F)), _full((1, F)),
            _full((F, 1)), _full((1, 1)),
        ],
        out_specs=_rows(N_TILE, 1),
        out_shape=jax.ShapeDtypeStruct((NP, 1), jnp.float32),
    )(V, w1, b1[None], w2, b2[None], w3, b3[None])


def kernel(node_pos, areas, edges, info, params):
    pos = node_pos[0]
    ar = areas[0]
    send = edges[0, :, 0]
    recv = edges[0, :, 1]
    inf = info.reshape(-1)

    fv, fe, gnn, dec = params["fv"], params["fe"], params["gnn"], params["dec"]

    n_tail = EP - N_EDGES - N_NODES
    all_send = jnp.concatenate([
        send, jnp.arange(N_NODES, dtype=send.dtype),
        jnp.full((n_tail,), N_NODES - 1, send.dtype)])
    all_recv = jnp.concatenate([recv, jnp.zeros((N_NODES + n_tail,), send.dtype)])
    realf = jnp.concatenate([
        jnp.ones((N_EDGES,), jnp.float32), jnp.zeros((N_NODES + n_tail,), jnp.float32)])
    order = jnp.argsort(all_send)
    ss = all_send[order]
    rs = all_recv[order]
    maskf = realf[order]
    los = ss[::CH]
    ss3 = ss.reshape(E_GRID, 1, E_TILE)
    mask3 = maskf.reshape(E_GRID, 1, E_TILE)

    (w1v, b1v), (w2v, b2v), (w3v, b3v) = fv["layers"]
    b1v_eff = b1v + inf @ w1v[4:12]
    x4 = jnp.concatenate([pos, ar], axis=1)
    x4 = jnp.pad(x4, ((0, NP - N_NODES), (0, 0)))
    V = _enc_mlp(x4, w1v[:4], b1v_eff, w2v, b2v, w3v, b3v,
                 fv["ln"][0], fv["ln"][1], N_TILE)

    ps = pos.at[ss].get(mode="promise_in_bounds")
    pr = pos.at[rs].get(mode="promise_in_bounds")
    d = ps - pr
    nrm = jnp.sqrt(jnp.sum(d * d, axis=-1, keepdims=True))
    ef = jnp.concatenate([d, nrm], axis=1)
    (w1e, b1e), (w2e, b2e), (w3e, b3e) = fe["layers"]
    E = _enc_mlp(ef, w1e, b1e, w2e, b2e, w3e, b3e,
                 fe["ln"][0], fe["ln"][1], E_TILE)

    g0 = gnn[0]
    w1_0 = g0["f_edge"]["layers"][0][0]
    Vs, Vr = _proj(V, w1_0[:F], w1_0[F:2 * F])
    for l in range(15):
        g = gnn[l]
        (we1, be1), (we2, be2), (we3, be3) = g["f_edge"]["layers"]
        Gr = Vr.at[rs].get(mode="promise_in_bounds")
        E, esum = _edge_layer(los, ss3, mask3, Gr, E, Vs,
                              we1[2 * F:], be1, we2, be2, we3, be3)
        (wn1, bn1), (wn2, bn2), (wn3, bn3) = g["f_node"]["layers"]
        if l + 1 < 15:
            w1_next = gnn[l + 1]["f_edge"]["layers"][0][0]
            a_next, b_next = w1_next[:F], w1_next[F:2 * F]
        else:
            a_next = jnp.zeros((F, F), jnp.float32)
            b_next = jnp.zeros((F, F), jnp.float32)
        V, Vs, Vr = _node_layer(V, esum, wn1[:F], wn1[F:], bn1,
                                wn2, bn2, wn3, bn3, a_next, b_next)

    (wd1, bd1), (wd2, bd2), (wd3, bd3) = dec["layers"]
    out = _dec(V, wd1, bd1, wd2, bd2, wd3, bd3)
    return out[:N_NODES][None]
